# baseline (speedup 1.0000x reference)
import jax
import jax.numpy as jnp
from jax import lax
from jax.experimental import pallas as pl
from jax.experimental.pallas import tpu as pltpu
from jax.experimental.pallas import tpu_sc as plsc

_N, _E, _F, _C = 10000, 320000, 128, 40
_FH = _F // 2
_NC, _NS = 2, 16
_CHUNK = 128
_CPT = 160
_NCHUNKS = _NS * _CPT
_EPAD = _NCHUNKS * _CHUNK - _E
_TRASH = 64
_ACCR = _N + _TRASH
_RPT0 = 632
_RPT1 = _N - 15 * _RPT0
_PH = 40


def _agg_body(x_hbm, src_hbm, dst_hbm, out_hbm,
              xtab, acc, srcbuf, dstbuf, rows0, rows1, sem0, sem1):
    cid = lax.axis_index("c")
    sid = lax.axis_index("s")

    @pl.when(sid < _NS - 1)
    def _():
        r0 = pl.multiple_of(sid * _RPT0, 8)
        pltpu.sync_copy(x_hbm.at[cid, pl.ds(r0, _RPT0)], xtab.at[pl.ds(r0, _RPT0)])
        pltpu.sync_copy(x_hbm.at[cid, pl.ds(r0, _RPT0)], acc.at[pl.ds(r0, _RPT0)])

    @pl.when(sid == _NS - 1)
    def _():
        pltpu.sync_copy(x_hbm.at[cid, pl.ds(15 * _RPT0, _RPT1)],
                        xtab.at[pl.ds(15 * _RPT0, _RPT1)])
        pltpu.sync_copy(x_hbm.at[cid, pl.ds(15 * _RPT0, _RPT1)],
                        acc.at[pl.ds(15 * _RPT0, _RPT1)])

    plsc.subcore_barrier()

    def _start(j, buf, sem):
        pltpu.make_async_copy(xtab.at[srcbuf.at[j]], buf, sem).start()

    def _wait(j, buf, sem):
        pltpu.make_async_copy(xtab.at[srcbuf.at[j]], buf, sem).wait()

    def _scat(j, buf):
        pltpu.sync_copy(buf, acc.at[dstbuf.at[j]], add=True)

    def _phase(off):
        start = pl.multiple_of(sid * _CPT + off, 8)
        pltpu.sync_copy(src_hbm.at[pl.ds(start, _PH)], srcbuf)
        pltpu.sync_copy(dst_hbm.at[pl.ds(start, _PH)], dstbuf)

        _start(0, rows0, sem0)

        def body(jj, carry):
            a = 2 * jj
            _start(a + 1, rows1, sem1)
            _wait(a, rows0, sem0)
            _scat(a, rows0)

            @pl.when(a + 2 < _PH)
            def _():
                _start(a + 2, rows0, sem0)

            _wait(a + 1, rows1, sem1)
            _scat(a + 1, rows1)
            return carry

        lax.fori_loop(0, _PH // 2, body, 0)

    for p in range(_CPT // _PH):
        _phase(p * _PH)

    plsc.subcore_barrier()

    @pl.when(sid < _NS - 1)
    def _():
        r0 = pl.multiple_of(sid * _RPT0, 8)
        pltpu.sync_copy(acc.at[pl.ds(r0, _RPT0)],
                        out_hbm.at[cid, pl.ds(r0, _RPT0)])

    @pl.when(sid == _NS - 1)
    def _():
        pltpu.sync_copy(acc.at[pl.ds(15 * _RPT0, _RPT1)],
                        out_hbm.at[cid, pl.ds(15 * _RPT0, _RPT1)])


def _agg(xs, src, dst):
    k = pl.kernel(
        _agg_body,
        mesh=plsc.VectorSubcoreMesh(core_axis_name="c", subcore_axis_name="s"),
        out_type=jax.ShapeDtypeStruct((_NC, _N, _FH), jnp.float32),
        scratch_types=[
            pltpu.VMEM_SHARED((_N, _FH), jnp.float32),
            pltpu.VMEM_SHARED((_ACCR, _FH), jnp.float32),
            pltpu.VMEM((_PH, _CHUNK), jnp.int32),
            pltpu.VMEM((_PH, _CHUNK), jnp.int32),
            pltpu.VMEM((_CHUNK, _FH), jnp.float32),
            pltpu.VMEM((_CHUNK, _FH), jnp.float32),
            pltpu.SemaphoreType.DMA,
            pltpu.SemaphoreType.DMA,
        ],
    )
    return k(xs, src, dst)


_RB = 2000
_GRID = _N // _RB


def _mlp_body(a_ref, w1_ref, b1_ref, gb_ref, w2_ref, b2_ref,
              o_ref, pool_ref, h_scr, sacc_ref, pacc_ref):
    p = pl.program_id(0)
    i = pl.program_id(1)

    @pl.when(p == 0)
    def _():
        hpre = jnp.concatenate([a_ref[0], a_ref[1]], axis=1)
        h = jnp.dot(hpre, w1_ref[...],
                    preferred_element_type=jnp.float32) + b1_ref[...]
        h_scr[pl.ds(i * _RB, _RB)] = h
        s1 = jnp.sum(h, axis=0, keepdims=True)
        s2 = jnp.sum(h * h, axis=0, keepdims=True)

        @pl.when(i == 0)
        def _():
            sacc_ref[...] = jnp.zeros_like(sacc_ref)

        sacc_ref[...] += jnp.concatenate([s1, s2], axis=0)

    @pl.when(p == 1)
    def _():
        mean = sacc_ref[0:1, :] * (1.0 / _N)
        var = sacc_ref[1:2, :] * (1.0 / _N) - mean * mean
        inv = lax.rsqrt(var + 1e-5)
        scale = gb_ref[0:1, :] * inv
        shift = gb_ref[1:2, :] - mean * scale
        hn = jnp.maximum(h_scr[pl.ds(i * _RB, _RB)] * scale + shift, 0.0)
        o = jnp.maximum(
            jnp.dot(hn, w2_ref[...], preferred_element_type=jnp.float32)
            + b2_ref[...], 0.0)
        o_ref[0] = o[:, :_FH]
        o_ref[1] = o[:, _FH:]

        @pl.when(i == 0)
        def _():
            pacc_ref[...] = jnp.zeros_like(pacc_ref)

        pacc_ref[...] += jnp.sum(o, axis=0, keepdims=True)

        @pl.when(i == _GRID - 1)
        def _():
            pool_ref[...] = pacc_ref[...]


def _mlp(a, w1, b1, g, be, w2, b2):
    gb = jnp.concatenate([g.reshape(1, _F), be.reshape(1, _F)], axis=0)
    return pl.pallas_call(
        _mlp_body,
        grid=(2, _GRID),
        in_specs=[
            pl.BlockSpec((_NC, _RB, _FH),
                         lambda p, i: (0, i * (1 - p) + (_GRID - 1) * p, 0)),
            pl.BlockSpec((_F, _F), lambda p, i: (0, 0)),
            pl.BlockSpec((1, _F), lambda p, i: (0, 0)),
            pl.BlockSpec((2, _F), lambda p, i: (0, 0)),
            pl.BlockSpec((_F, _F), lambda p, i: (0, 0)),
            pl.BlockSpec((1, _F), lambda p, i: (0, 0)),
        ],
        out_specs=[
            pl.BlockSpec((_NC, _RB, _FH), lambda p, i: (0, i * p, 0)),
            pl.BlockSpec((1, _F), lambda p, i: (0, 0)),
        ],
        out_shape=[
            jax.ShapeDtypeStruct((_NC, _N, _FH), jnp.float32),
            jax.ShapeDtypeStruct((1, _F), jnp.float32),
        ],
        scratch_shapes=[
            pltpu.VMEM((_N, _F), jnp.float32),
            pltpu.VMEM((2, _F), jnp.float32),
            pltpu.VMEM((1, _F), jnp.float32),
        ],
    )(a, w1, b1.reshape(1, _F), gb, w2, b2.reshape(1, _F))


def _head_body(p0_ref, p1_ref, p2_ref, w_ref, b_ref, o_ref):
    o_ref[...] = (
        jnp.dot(p0_ref[...], w_ref[0:_F], preferred_element_type=jnp.float32)
        + jnp.dot(p1_ref[...], w_ref[_F:2 * _F], preferred_element_type=jnp.float32)
        + jnp.dot(p2_ref[...], w_ref[2 * _F:3 * _F], preferred_element_type=jnp.float32)
        + b_ref[...])


def _head(p0, p1, p2, wll, bll):
    return pl.pallas_call(
        _head_body,
        out_shape=jax.ShapeDtypeStruct((1, _C), jnp.float32),
    )(p0, p1, p2, wll, bll.reshape(1, _C))


def kernel(x, edge_idx, W1_0, b1_0, g_0, be_0, W2_0, b2_0,
           W1_1, b1_1, g_1, be_1, W2_1, b2_1,
           W1_2, b1_2, g_2, be_2, W2_2, b2_2, Wll, bll):
    src = jnp.concatenate(
        [edge_idx[0], jnp.zeros((_EPAD,), jnp.int32)]).reshape(_NCHUNKS, _CHUNK)
    dst = jnp.concatenate(
        [edge_idx[1], _N + (jnp.arange(_EPAD, dtype=jnp.int32) % _TRASH)]
    ).reshape(_NCHUNKS, _CHUNK)
    layers = ((W1_0, b1_0, g_0, be_0, W2_0, b2_0),
              (W1_1, b1_1, g_1, be_1, W2_1, b2_1),
              (W1_2, b1_2, g_2, be_2, W2_2, b2_2))
    hs = jnp.stack([x[:, :_FH], x[:, _FH:]])
    pools = []
    for (W1, b1, g, be, W2, b2) in layers:
        a = _agg(hs, src, dst)
        hs, pool = _mlp(a, W1, b1, g, be, W2, b2)
        pools.append(pool)
    return _head(pools[0], pools[1], pools[2], Wll, bll)


# speedup vs baseline: 1.0346x; 1.0346x over previous
import jax
import jax.numpy as jnp
from jax import lax
from jax.experimental import pallas as pl
from jax.experimental.pallas import tpu as pltpu
from jax.experimental.pallas import tpu_sc as plsc

_N, _E, _F, _C = 10000, 320000, 128, 40
_FH = _F // 2
_NC, _NS = 2, 16
_CHUNK = 64
_CPT = 320
_NCHUNKS = _NS * _CPT
_EPAD = _NCHUNKS * _CHUNK - _E
_TRASH = 64
_ACCR = _N + _TRASH
_RPT0 = 632
_RPT1 = _N - 15 * _RPT0
_PH = 80


def _agg_body(x_hbm, src_hbm, dst_hbm, out_hbm,
              xtab, acc, srcbuf, dstbuf, rows0, rows1, sem0, sem1):
    cid = lax.axis_index("c")
    sid = lax.axis_index("s")

    @pl.when(sid < _NS - 1)
    def _():
        r0 = pl.multiple_of(sid * _RPT0, 8)
        pltpu.sync_copy(x_hbm.at[cid, pl.ds(r0, _RPT0)], xtab.at[pl.ds(r0, _RPT0)])
        pltpu.sync_copy(x_hbm.at[cid, pl.ds(r0, _RPT0)], acc.at[pl.ds(r0, _RPT0)])

    @pl.when(sid == _NS - 1)
    def _():
        pltpu.sync_copy(x_hbm.at[cid, pl.ds(15 * _RPT0, _RPT1)],
                        xtab.at[pl.ds(15 * _RPT0, _RPT1)])
        pltpu.sync_copy(x_hbm.at[cid, pl.ds(15 * _RPT0, _RPT1)],
                        acc.at[pl.ds(15 * _RPT0, _RPT1)])

    plsc.subcore_barrier()

    def _start(j, buf, sem):
        pltpu.make_async_copy(xtab.at[srcbuf.at[j]], buf, sem).start()

    def _wait(j, buf, sem):
        pltpu.make_async_copy(xtab.at[srcbuf.at[j]], buf, sem).wait()

    def _scat(j, buf):
        pltpu.sync_copy(buf, acc.at[dstbuf.at[j]], add=True)

    def _phase(off):
        start = pl.multiple_of(sid * _CPT + off, 8)
        pltpu.sync_copy(src_hbm.at[pl.ds(start, _PH)], srcbuf)
        pltpu.sync_copy(dst_hbm.at[pl.ds(start, _PH)], dstbuf)

        _start(0, rows0, sem0)

        def body(jj, carry):
            a = 2 * jj
            _start(a + 1, rows1, sem1)
            _wait(a, rows0, sem0)
            _scat(a, rows0)

            @pl.when(a + 2 < _PH)
            def _():
                _start(a + 2, rows0, sem0)

            _wait(a + 1, rows1, sem1)
            _scat(a + 1, rows1)
            return carry

        lax.fori_loop(0, _PH // 2, body, 0)

    for p in range(_CPT // _PH):
        _phase(p * _PH)

    plsc.subcore_barrier()

    @pl.when(sid < _NS - 1)
    def _():
        r0 = pl.multiple_of(sid * _RPT0, 8)
        pltpu.sync_copy(acc.at[pl.ds(r0, _RPT0)],
                        out_hbm.at[cid, pl.ds(r0, _RPT0)])

    @pl.when(sid == _NS - 1)
    def _():
        pltpu.sync_copy(acc.at[pl.ds(15 * _RPT0, _RPT1)],
                        out_hbm.at[cid, pl.ds(15 * _RPT0, _RPT1)])


def _agg(xs, src, dst):
    k = pl.kernel(
        _agg_body,
        mesh=plsc.VectorSubcoreMesh(core_axis_name="c", subcore_axis_name="s"),
        out_type=jax.ShapeDtypeStruct((_NC, _N, _FH), jnp.float32),
        scratch_types=[
            pltpu.VMEM_SHARED((_N, _FH), jnp.float32),
            pltpu.VMEM_SHARED((_ACCR, _FH), jnp.float32),
            pltpu.VMEM((_PH, _CHUNK), jnp.int32),
            pltpu.VMEM((_PH, _CHUNK), jnp.int32),
            pltpu.VMEM((_CHUNK, _FH), jnp.float32),
            pltpu.VMEM((_CHUNK, _FH), jnp.float32),
            pltpu.SemaphoreType.DMA,
            pltpu.SemaphoreType.DMA,
        ],
    )
    return k(xs, src, dst)


_RB = 2000
_GRID = _N // _RB


def _mlp_body(a_ref, w1_ref, b1_ref, gb_ref, w2_ref, b2_ref,
              o_ref, pool_ref, h_scr, sacc_ref, pacc_ref):
    p = pl.program_id(0)
    i = pl.program_id(1)

    @pl.when(p == 0)
    def _():
        hpre = jnp.concatenate([a_ref[0], a_ref[1]], axis=1)
        h = jnp.dot(hpre, w1_ref[...],
                    preferred_element_type=jnp.float32) + b1_ref[...]
        h_scr[pl.ds(i * _RB, _RB)] = h
        s1 = jnp.sum(h, axis=0, keepdims=True)
        s2 = jnp.sum(h * h, axis=0, keepdims=True)

        @pl.when(i == 0)
        def _():
            sacc_ref[...] = jnp.zeros_like(sacc_ref)

        sacc_ref[...] += jnp.concatenate([s1, s2], axis=0)

    @pl.when(p == 1)
    def _():
        mean = sacc_ref[0:1, :] * (1.0 / _N)
        var = sacc_ref[1:2, :] * (1.0 / _N) - mean * mean
        inv = lax.rsqrt(var + 1e-5)
        scale = gb_ref[0:1, :] * inv
        shift = gb_ref[1:2, :] - mean * scale
        hn = jnp.maximum(h_scr[pl.ds(i * _RB, _RB)] * scale + shift, 0.0)
        o = jnp.maximum(
            jnp.dot(hn, w2_ref[...], preferred_element_type=jnp.float32)
            + b2_ref[...], 0.0)
        o_ref[0] = o[:, :_FH]
        o_ref[1] = o[:, _FH:]

        @pl.when(i == 0)
        def _():
            pacc_ref[...] = jnp.zeros_like(pacc_ref)

        pacc_ref[...] += jnp.sum(o, axis=0, keepdims=True)

        @pl.when(i == _GRID - 1)
        def _():
            pool_ref[...] = pacc_ref[...]


def _mlp(a, w1, b1, g, be, w2, b2):
    gb = jnp.concatenate([g.reshape(1, _F), be.reshape(1, _F)], axis=0)
    return pl.pallas_call(
        _mlp_body,
        grid=(2, _GRID),
        in_specs=[
            pl.BlockSpec((_NC, _RB, _FH),
                         lambda p, i: (0, i * (1 - p) + (_GRID - 1) * p, 0)),
            pl.BlockSpec((_F, _F), lambda p, i: (0, 0)),
            pl.BlockSpec((1, _F), lambda p, i: (0, 0)),
            pl.BlockSpec((2, _F), lambda p, i: (0, 0)),
            pl.BlockSpec((_F, _F), lambda p, i: (0, 0)),
            pl.BlockSpec((1, _F), lambda p, i: (0, 0)),
        ],
        out_specs=[
            pl.BlockSpec((_NC, _RB, _FH), lambda p, i: (0, i * p, 0)),
            pl.BlockSpec((1, _F), lambda p, i: (0, 0)),
        ],
        out_shape=[
            jax.ShapeDtypeStruct((_NC, _N, _FH), jnp.float32),
            jax.ShapeDtypeStruct((1, _F), jnp.float32),
        ],
        scratch_shapes=[
            pltpu.VMEM((_N, _F), jnp.float32),
            pltpu.VMEM((2, _F), jnp.float32),
            pltpu.VMEM((1, _F), jnp.float32),
        ],
    )(a, w1, b1.reshape(1, _F), gb, w2, b2.reshape(1, _F))


def _head_body(p0_ref, p1_ref, p2_ref, w_ref, b_ref, o_ref):
    o_ref[...] = (
        jnp.dot(p0_ref[...], w_ref[0:_F], preferred_element_type=jnp.float32)
        + jnp.dot(p1_ref[...], w_ref[_F:2 * _F], preferred_element_type=jnp.float32)
        + jnp.dot(p2_ref[...], w_ref[2 * _F:3 * _F], preferred_element_type=jnp.float32)
        + b_ref[...])


def _head(p0, p1, p2, wll, bll):
    return pl.pallas_call(
        _head_body,
        out_shape=jax.ShapeDtypeStruct((1, _C), jnp.float32),
    )(p0, p1, p2, wll, bll.reshape(1, _C))


def kernel(x, edge_idx, W1_0, b1_0, g_0, be_0, W2_0, b2_0,
           W1_1, b1_1, g_1, be_1, W2_1, b2_1,
           W1_2, b1_2, g_2, be_2, W2_2, b2_2, Wll, bll):
    src = jnp.concatenate(
        [edge_idx[0], jnp.zeros((_EPAD,), jnp.int32)]).reshape(_NCHUNKS, _CHUNK)
    dst = jnp.concatenate(
        [edge_idx[1], _N + (jnp.arange(_EPAD, dtype=jnp.int32) % _TRASH)]
    ).reshape(_NCHUNKS, _CHUNK)
    layers = ((W1_0, b1_0, g_0, be_0, W2_0, b2_0),
              (W1_1, b1_1, g_1, be_1, W2_1, b2_1),
              (W1_2, b1_2, g_2, be_2, W2_2, b2_2))
    hs = jnp.stack([x[:, :_FH], x[:, _FH:]])
    pools = []
    for (W1, b1, g, be, W2, b2) in layers:
        a = _agg(hs, src, dst)
        hs, pool = _mlp(a, W1, b1, g, be, W2, b2)
        pools.append(pool)
    return _head(pools[0], pools[1], pools[2], Wll, bll)
